# baseline (speedup 1.0000x reference)
import jax
import jax.numpy as jnp
from jax import lax
from jax.experimental import pallas as pl
from jax.experimental.pallas import tpu as pltpu

NUM_TOKENS = 32768
D_MODEL = 4096
NUM_EXPERTS = 64

TM = 1024
KSPLIT = 4
KD = D_MODEL // KSPLIT


def _router_kernel(*refs):
    x_refs = refs[:KSPLIT]
    wt_ref = refs[KSPLIT]
    oh_ref, top_ref, logits_ref = refs[KSPLIT + 1:]
    logits = jnp.dot(
        x_refs[0][...], wt_ref[pl.ds(0, KD), :],
        preferred_element_type=jnp.float32,
    )
    for k in range(1, KSPLIT):
        logits += jnp.dot(
            x_refs[k][...], wt_ref[pl.ds(k * KD, KD), :],
            preferred_element_type=jnp.float32,
        )
    m = jnp.max(logits, axis=1, keepdims=True)
    s = jnp.sum(jnp.exp(logits - m), axis=1, keepdims=True)
    ii = lax.broadcasted_iota(jnp.int32, logits.shape, 1)
    cand = jnp.where(logits == m, ii, NUM_EXPERTS)
    first = jnp.min(cand, axis=1, keepdims=True)
    oh_ref[...] = (ii == first).astype(jnp.int32)
    top_ref[...] = 1.0 / s
    logits_ref[...] = logits


@jax.jit
def kernel(x, W):
    wt = W.T
    grid = (NUM_TOKENS // TM,)

    def x_spec(k):
        return pl.BlockSpec((TM, KD), lambda i, k=k: (i, k))

    oh, top, logits = pl.pallas_call(
        _router_kernel,
        grid=grid,
        in_specs=[x_spec(k) for k in range(KSPLIT)]
        + [pl.BlockSpec((D_MODEL, NUM_EXPERTS), lambda i: (0, 0))],
        out_specs=[
            pl.BlockSpec((TM, NUM_EXPERTS), lambda i: (i, 0)),
            pl.BlockSpec((TM, 1), lambda i: (i, 0)),
            pl.BlockSpec((TM, NUM_EXPERTS), lambda i: (i, 0)),
        ],
        out_shape=[
            jax.ShapeDtypeStruct((NUM_TOKENS, NUM_EXPERTS), jnp.int32),
            jax.ShapeDtypeStruct((NUM_TOKENS, 1), jnp.float32),
            jax.ShapeDtypeStruct((NUM_TOKENS, NUM_EXPERTS), jnp.float32),
        ],
        compiler_params=pltpu.CompilerParams(
            dimension_semantics=("arbitrary",),
        ),
    )(*([x] * KSPLIT + [wt]))
    return oh, top, logits


# speedup vs baseline: 1.0203x; 1.0203x over previous
import jax
import jax.numpy as jnp
from jax import lax
from jax.experimental import pallas as pl
from jax.experimental.pallas import tpu as pltpu

NUM_TOKENS = 32768
D_MODEL = 4096
NUM_EXPERTS = 64

TM = 1024


def _router_kernel(x_ref, wt_ref, oh_ref, top_ref, logits_ref):
    logits = jnp.dot(x_ref[...], wt_ref[...], preferred_element_type=jnp.float32)
    m = jnp.max(logits, axis=1, keepdims=True)
    s = jnp.sum(jnp.exp(logits - m), axis=1, keepdims=True)
    ii = lax.broadcasted_iota(jnp.int32, logits.shape, 1)
    cand = jnp.where(logits == m, ii, NUM_EXPERTS)
    first = jnp.min(cand, axis=1, keepdims=True)
    oh_ref[...] = (ii == first).astype(jnp.int32)
    top_ref[...] = (1.0 / s)[:, 0]
    logits_ref[...] = logits


@jax.jit
def kernel(x, W):
    wt = W.T
    grid = (NUM_TOKENS // TM,)
    oh, top, logits = pl.pallas_call(
        _router_kernel,
        grid=grid,
        in_specs=[
            pl.BlockSpec((TM, D_MODEL), lambda i: (i, 0)),
            pl.BlockSpec((D_MODEL, NUM_EXPERTS), lambda i: (0, 0)),
        ],
        out_specs=[
            pl.BlockSpec((TM, NUM_EXPERTS), lambda i: (i, 0)),
            pl.BlockSpec((TM,), lambda i: (i,)),
            pl.BlockSpec((TM, NUM_EXPERTS), lambda i: (i, 0)),
        ],
        out_shape=[
            jax.ShapeDtypeStruct((NUM_TOKENS, NUM_EXPERTS), jnp.int32),
            jax.ShapeDtypeStruct((NUM_TOKENS,), jnp.float32),
            jax.ShapeDtypeStruct((NUM_TOKENS, NUM_EXPERTS), jnp.float32),
        ],
        compiler_params=pltpu.CompilerParams(
            dimension_semantics=("arbitrary",),
        ),
    )(x, wt)
    return oh, top.reshape(NUM_TOKENS, 1), logits


# speedup vs baseline: 1.0722x; 1.0509x over previous
import jax
import jax.numpy as jnp
from jax import lax
from jax.experimental import pallas as pl
from jax.experimental.pallas import tpu as pltpu

NUM_TOKENS = 32768
D_MODEL = 4096
NUM_EXPERTS = 64

TM = 1024


RSPLIT = 4
TR = TM // RSPLIT


def _router_kernel(*refs):
    x_refs = refs[:RSPLIT]
    wt_ref = refs[RSPLIT]
    oh_ref, top_ref, logits_ref = refs[RSPLIT + 1:]
    logits = jnp.concatenate(
        [jnp.dot(xr[...], wt_ref[...], preferred_element_type=jnp.float32)
         for xr in x_refs],
        axis=0,
    )
    m = jnp.max(logits, axis=1, keepdims=True)
    s = jnp.sum(jnp.exp(logits - m), axis=1, keepdims=True)
    ii = lax.broadcasted_iota(jnp.int32, logits.shape, 1)
    cand = jnp.where(logits == m, ii, NUM_EXPERTS)
    first = jnp.min(cand, axis=1, keepdims=True)
    oh_ref[...] = (ii == first).astype(jnp.int32)
    top_ref[...] = (1.0 / s)[:, 0]
    logits_ref[...] = logits


@jax.jit
def kernel(x, W):
    wt = W.T
    grid = (NUM_TOKENS // TM,)
    oh, top, logits = pl.pallas_call(
        _router_kernel,
        grid=grid,
        in_specs=[
            pl.BlockSpec((TR, D_MODEL), lambda i, r=r: (i * RSPLIT + r, 0))
            for r in range(RSPLIT)
        ]
        + [pl.BlockSpec((D_MODEL, NUM_EXPERTS), lambda i: (0, 0))],
        out_specs=[
            pl.BlockSpec((TM, NUM_EXPERTS), lambda i: (i, 0)),
            pl.BlockSpec((TM,), lambda i: (i,)),
            pl.BlockSpec((TM, NUM_EXPERTS), lambda i: (i, 0)),
        ],
        out_shape=[
            jax.ShapeDtypeStruct((NUM_TOKENS, NUM_EXPERTS), jnp.int32),
            jax.ShapeDtypeStruct((NUM_TOKENS,), jnp.float32),
            jax.ShapeDtypeStruct((NUM_TOKENS, NUM_EXPERTS), jnp.float32),
        ],
        compiler_params=pltpu.CompilerParams(
            dimension_semantics=("parallel",),
        ),
    )(*([x] * RSPLIT + [wt]))
    return oh, top.reshape(NUM_TOKENS, 1), logits


# speedup vs baseline: 1.1006x; 1.0265x over previous
import jax
import jax.numpy as jnp
from jax import lax
from jax.experimental import pallas as pl
from jax.experimental.pallas import tpu as pltpu

NUM_TOKENS = 32768
D_MODEL = 4096
NUM_EXPERTS = 64

TM = 1024


RSPLIT = 8
TR = TM // RSPLIT


def _router_kernel(*refs):
    x_refs = refs[:RSPLIT]
    wt_ref = refs[RSPLIT]
    oh_ref, top_ref, logits_ref = refs[RSPLIT + 1:]
    logits = jnp.concatenate(
        [jnp.dot(xr[...], wt_ref[...], preferred_element_type=jnp.float32)
         for xr in x_refs],
        axis=0,
    )
    m = jnp.max(logits, axis=1, keepdims=True)
    s = jnp.sum(jnp.exp(logits - m), axis=1, keepdims=True)
    ii = lax.broadcasted_iota(jnp.int32, logits.shape, 1)
    cand = jnp.where(logits == m, ii, NUM_EXPERTS)
    first = jnp.min(cand, axis=1, keepdims=True)
    oh_ref[...] = (ii == first).astype(jnp.int32)
    top_ref[...] = (1.0 / s)[:, 0]
    logits_ref[...] = logits


@jax.jit
def kernel(x, W):
    wt = W.T
    grid = (NUM_TOKENS // TM,)
    oh, top, logits = pl.pallas_call(
        _router_kernel,
        grid=grid,
        in_specs=[
            pl.BlockSpec((TR, D_MODEL), lambda i, r=r: (i * RSPLIT + r, 0))
            for r in range(RSPLIT)
        ]
        + [pl.BlockSpec((D_MODEL, NUM_EXPERTS), lambda i: (0, 0))],
        out_specs=[
            pl.BlockSpec((TM, NUM_EXPERTS), lambda i: (i, 0)),
            pl.BlockSpec((TM,), lambda i: (i,)),
            pl.BlockSpec((TM, NUM_EXPERTS), lambda i: (i, 0)),
        ],
        out_shape=[
            jax.ShapeDtypeStruct((NUM_TOKENS, NUM_EXPERTS), jnp.int32),
            jax.ShapeDtypeStruct((NUM_TOKENS,), jnp.float32),
            jax.ShapeDtypeStruct((NUM_TOKENS, NUM_EXPERTS), jnp.float32),
        ],
        compiler_params=pltpu.CompilerParams(
            dimension_semantics=("parallel",),
        ),
    )(*([x] * RSPLIT + [wt]))
    return oh, top.reshape(NUM_TOKENS, 1), logits
